# Initial kernel scaffold; baseline (speedup 1.0000x reference)
import jax
import jax.numpy as jnp
from jax.experimental import pallas as pl


def kernel(output, target):
    raise NotImplementedError("write your pallas kernel here")


# speedup vs baseline: 9.8119x; 9.8119x over previous
import functools
import math

import jax
import jax.numpy as jnp
from jax.experimental import pallas as pl

V = 32000
N = 4096
_SMOOTH = 0.1 / (V - 2)
_CONF = 0.9
_C_ENT = (V - 2) * _SMOOTH * math.log(_SMOOTH) + _CONF * math.log(_CONF)

_BC = 1280


def _loss_body(x_ref, t_ref, out_ref):
    j = pl.program_id(0)
    x = x_ref[...]
    t = t_ref[...]
    col = jax.lax.broadcasted_iota(jnp.int32, (N, _BC), 1) + j * _BC
    validf = jnp.where(t != 0, 1.0, 0.0)
    w = jnp.where(col == 0, 0.0, _SMOOTH) + jnp.where(col == t, _CONF - _SMOOTH, 0.0)
    partial = -jnp.sum(x * (w * validf), keepdims=True)

    @pl.when(j == 0)
    def _init():
        out_ref[...] = _C_ENT * jnp.sum(validf, keepdims=True)

    out_ref[...] += partial


@functools.partial(jax.jit, static_argnames=())
def kernel(output, target):
    t2 = target.astype(jnp.int32).reshape(N, 1)
    res = pl.pallas_call(
        _loss_body,
        grid=(V // _BC,),
        in_specs=[
            pl.BlockSpec((N, _BC), lambda j: (0, j)),
            pl.BlockSpec((N, 1), lambda j: (0, 0)),
        ],
        out_specs=pl.BlockSpec((1, 1), lambda j: (0, 0)),
        out_shape=jax.ShapeDtypeStruct((1, 1), jnp.float32),
    )(output, t2)
    return res[0, 0]
